# Initial kernel scaffold; baseline (speedup 1.0000x reference)
import jax
import jax.numpy as jnp
from jax.experimental import pallas as pl


def kernel(x, edge_index, edge_type, W, W_loop, bias):
    raise NotImplementedError("write your pallas kernel here")


# speedup vs baseline: 106.0819x; 106.0819x over previous
import functools

import jax
import jax.numpy as jnp
from jax import lax
from jax.experimental import pallas as pl
from jax.experimental.pallas import tpu as pltpu
from jax.experimental.pallas import tpu_sc as plsc

N = 10000
E = 320000
D = 128
R = 4
NPAD = 10240
NW = 32
E_PER_W = E // NW
HIST = R * NPAD

_mesh = plsc.VectorSubcoreMesh(core_axis_name="c", subcore_axis_name="s")


@functools.partial(
    pl.kernel,
    out_type=jax.ShapeDtypeStruct((NW * HIST,), jnp.float32),
    mesh=_mesh,
    compiler_params=pltpu.CompilerParams(needs_layout_passes=False),
    scratch_types=[
        pltpu.VMEM((HIST,), jnp.float32),
        pltpu.VMEM((E_PER_W,), jnp.int32),
        pltpu.VMEM((E_PER_W,), jnp.int32),
    ],
)
def _sc_histogram(src_hbm, et_hbm, out_hbm, hist_v, src_v, et_v):
    wid = lax.axis_index("s") * 2 + lax.axis_index("c")
    base = wid * E_PER_W

    zeros = jnp.zeros((16,), jnp.float32)

    def zero_body(i, _):
        for j in range(8):
            hist_v[pl.ds((i * 8 + j) * 16, 16)] = zeros
        return _

    lax.fori_loop(0, HIST // (16 * 8), zero_body, None)

    pltpu.sync_copy(src_hbm.at[pl.ds(base, E_PER_W)], src_v)
    pltpu.sync_copy(et_hbm.at[pl.ds(base, E_PER_W)], et_v)

    ones = jnp.ones((16,), jnp.float32)

    def edge_body(i, _):
        for j in range(5):
            off = (i * 5 + j) * 16
            s = src_v[pl.ds(off, 16)]
            t = et_v[pl.ds(off, 16)]
            plsc.addupdate_scatter(hist_v, [t * NPAD + s], ones)
        return _

    lax.fori_loop(0, E_PER_W // (16 * 5), edge_body, None)

    pltpu.sync_copy(hist_v, out_hbm.at[pl.ds(wid * HIST, HIST)])


def _tc_body(c_ref, x_ref, w_ref, wl_ref, b_ref, o_ref):
    counts = c_ref[...]
    xp = x_ref[...]
    acc = jnp.dot(counts, xp, preferred_element_type=jnp.float32)
    g = lax.broadcasted_iota(jnp.int32, (R, NW * R), 1)
    r = lax.broadcasted_iota(jnp.int32, (R, NW * R), 0)
    sel = (g % R == r).astype(jnp.float32)
    s4 = jnp.dot(sel, acc, preferred_element_type=jnp.float32)
    colsum = jnp.sum(xp, axis=0, keepdims=True)
    out = jnp.dot(colsum, wl_ref[...], preferred_element_type=jnp.float32)
    for rr in range(R):
        out = out + jnp.dot(s4[rr:rr + 1, :], w_ref[rr],
                            preferred_element_type=jnp.float32)
    o_ref[...] = out * (1.0 / N) + b_ref[...]


_tc_final = pl.pallas_call(
    _tc_body,
    out_shape=jax.ShapeDtypeStruct((1, D), jnp.float32),
)


@jax.jit
def kernel(x, edge_index, edge_type, W, W_loop, bias):
    src = edge_index[0]
    counts_flat = _sc_histogram(src, edge_type)
    counts = counts_flat.reshape(NW * R, NPAD)
    xpad = jnp.zeros((NPAD, D), jnp.float32).at[:N].set(x)
    return _tc_final(counts, xpad, W, W_loop, bias.reshape(1, D))


# speedup vs baseline: 125.6912x; 1.1849x over previous
import functools

import jax
import jax.numpy as jnp
from jax import lax
from jax.experimental import pallas as pl
from jax.experimental.pallas import tpu as pltpu
from jax.experimental.pallas import tpu_sc as plsc

N = 10000
E = 320000
D = 128
R = 4
NW = 32
E_PER_W = E // NW
P = R // 2

_mesh = plsc.VectorSubcoreMesh(core_axis_name="c", subcore_axis_name="s")


@functools.partial(
    pl.kernel,
    out_type=jax.ShapeDtypeStruct((NW * P, N), jnp.int32),
    mesh=_mesh,
    compiler_params=pltpu.CompilerParams(needs_layout_passes=False),
    scratch_types=[
        pltpu.VMEM((P, N), jnp.int32),
        pltpu.VMEM((E_PER_W,), jnp.int32),
        pltpu.VMEM((E_PER_W,), jnp.int32),
    ],
)
def _sc_histogram(src_hbm, et_hbm, out_hbm, hist_v, src_v, et_v):
    wid = lax.axis_index("s") * 2 + lax.axis_index("c")
    base = wid * E_PER_W

    zeros = jnp.zeros((16,), jnp.int32)

    def zero_body(i, _):
        for j in range(5):
            off = (i * 5 + j) * 16
            hist_v[0, pl.ds(off, 16)] = zeros
            hist_v[1, pl.ds(off, 16)] = zeros
        return _

    lax.fori_loop(0, N // (16 * 5), zero_body, None)

    pltpu.sync_copy(src_hbm.at[pl.ds(base, E_PER_W)], src_v)
    pltpu.sync_copy(et_hbm.at[pl.ds(base, E_PER_W)], et_v)

    one = jnp.ones((16,), jnp.int32)
    hi_one = jnp.full((16,), 1 << 16, jnp.int32)
    two = jnp.full((16,), 2, jnp.int32)

    def edge_body(i, _):
        for j in range(5):
            off = (i * 5 + j) * 16
            s = src_v[pl.ds(off, 16)]
            t = et_v[pl.ds(off, 16)]
            val = jnp.where(t >= two, hi_one, one)
            plsc.addupdate_scatter(hist_v, [t & one, s], val)
        return _

    lax.fori_loop(0, E_PER_W // (16 * 5), edge_body, None)

    pltpu.sync_copy(hist_v, out_hbm.at[pl.ds(wid * P, P)])


def _tc_body(c_ref, x_ref, w_ref, b_ref, o_ref):
    c = c_ref[...]
    low = (c & 0xFFFF).astype(jnp.float32)
    high = (c >> 16).astype(jnp.float32)
    g = lax.broadcasted_iota(jnp.int32, (P, NW * P), 1)
    p = lax.broadcasted_iota(jnp.int32, (P, NW * P), 0)
    sel = (g % P == p).astype(jnp.float32)
    c01 = jnp.dot(sel, low, preferred_element_type=jnp.float32)
    c23 = jnp.dot(sel, high, preferred_element_type=jnp.float32)
    c4 = jnp.concatenate([c01, c23], axis=0)
    x = x_ref[...]
    s4 = jnp.dot(c4, x, preferred_element_type=jnp.float32)
    colsum = jnp.sum(x, axis=0, keepdims=True)
    all5 = jnp.concatenate([s4, colsum], axis=0)
    out = jnp.zeros((1, D), jnp.float32)
    for rr in range(R + 1):
        out = out + jnp.dot(all5[rr:rr + 1, :], w_ref[rr],
                            preferred_element_type=jnp.float32)
    o_ref[...] = out * (1.0 / N) + b_ref[...]


_tc_final = pl.pallas_call(
    _tc_body,
    out_shape=jax.ShapeDtypeStruct((1, D), jnp.float32),
)


@jax.jit
def kernel(x, edge_index, edge_type, W, W_loop, bias):
    counts = _sc_histogram(edge_index[0], edge_type)
    w5 = jnp.concatenate([W, W_loop[None]], axis=0)
    return _tc_final(counts, x, w5, bias.reshape(1, D))


# speedup vs baseline: 177.8487x; 1.4150x over previous
import functools

import jax
import jax.numpy as jnp
from jax import lax
from jax.experimental import pallas as pl
from jax.experimental.pallas import tpu as pltpu
from jax.experimental.pallas import tpu_sc as plsc

N = 10000
E = 320000
D = 128
R = 4
NW = 32
P = R // 2
ETILES = E // 128
WTILES = ETILES // NW + 1
WEDGE = WTILES * 128

_mesh = plsc.VectorSubcoreMesh(core_axis_name="c", subcore_axis_name="s")


@functools.partial(
    pl.kernel,
    out_type=jax.ShapeDtypeStruct((NW * P, N), jnp.int32),
    mesh=_mesh,
    compiler_params=pltpu.CompilerParams(needs_layout_passes=False),
    scratch_types=[
        pltpu.VMEM((P, N), jnp.int32),
        pltpu.VMEM((2, WEDGE), jnp.int32),
        pltpu.VMEM((WEDGE,), jnp.int32),
        pltpu.SemaphoreType.DMA,
        pltpu.SemaphoreType.DMA,
    ],
)
def _sc_histogram(ei_hbm, et_hbm, out_hbm, hist_v, ei_v, et_v, sem1, sem2):
    wid = lax.axis_index("s") * 2 + lax.axis_index("c")
    t0 = (wid * ETILES) // NW
    t1 = ((wid + 1) * ETILES) // NW
    start = jnp.minimum(t0, ETILES - WTILES)
    doff = (t0 - start) * 128
    n_it = t1 - t0

    cp1 = pltpu.async_copy(
        ei_hbm.at[:, pl.ds(start * 128, WEDGE)], ei_v, sem1)
    cp2 = pltpu.async_copy(
        et_hbm.at[pl.ds(start * 128, WEDGE)], et_v, sem2)

    zeros = jnp.zeros((16,), jnp.int32)

    def zero_body(i, _):
        for j in range(5):
            off = (i * 5 + j) * 16
            hist_v[0, pl.ds(off, 16)] = zeros
            hist_v[1, pl.ds(off, 16)] = zeros
        return _

    lax.fori_loop(0, N // (16 * 5), zero_body, None)

    cp1.wait()
    cp2.wait()

    one = jnp.ones((16,), jnp.int32)
    hi_one = jnp.full((16,), 1 << 16, jnp.int32)
    two = jnp.full((16,), 2, jnp.int32)

    def edge_body(i, _):
        base = doff + i * 128
        for j in range(8):
            off = base + j * 16
            s = ei_v[0, pl.ds(off, 16)]
            t = et_v[pl.ds(off, 16)]
            val = jnp.where(t >= two, hi_one, one)
            plsc.addupdate_scatter(hist_v, [t & one, s], val)
        return _

    lax.fori_loop(0, n_it, edge_body, None)

    pltpu.sync_copy(hist_v, out_hbm.at[pl.ds(wid * P, P)])


def _tc_body(c_ref, x_ref, w_ref, wl_ref, b_ref, o_ref):
    c = c_ref[...]
    low = (c & 0xFFFF).astype(jnp.float32)
    high = (c >> 16).astype(jnp.float32)
    g = lax.broadcasted_iota(jnp.int32, (P, NW * P), 1)
    p = lax.broadcasted_iota(jnp.int32, (P, NW * P), 0)
    sel = (g % P == p).astype(jnp.float32)
    c01 = jnp.dot(sel, low, preferred_element_type=jnp.float32)
    c23 = jnp.dot(sel, high, preferred_element_type=jnp.float32)
    c4 = jnp.concatenate([c01, c23], axis=0)
    x = x_ref[...]
    s4 = jnp.dot(c4, x, preferred_element_type=jnp.float32)
    colsum = jnp.sum(x, axis=0, keepdims=True)
    out = jnp.dot(colsum, wl_ref[...], preferred_element_type=jnp.float32)
    for rr in range(R):
        out = out + jnp.dot(s4[rr:rr + 1, :], w_ref[rr],
                            preferred_element_type=jnp.float32)
    o_ref[...] = out * (1.0 / N) + b_ref[...]


_tc_final = pl.pallas_call(
    _tc_body,
    out_shape=jax.ShapeDtypeStruct((1, D), jnp.float32),
)


@jax.jit
def kernel(x, edge_index, edge_type, W, W_loop, bias):
    counts = _sc_histogram(edge_index, edge_type)
    return _tc_final(counts, x, W, W_loop, bias.reshape(1, D))
